# Initial kernel scaffold; baseline (speedup 1.0000x reference)
import jax
import jax.numpy as jnp
from jax.experimental import pallas as pl


def kernel(x, w_gate, W, b):
    raise NotImplementedError("write your pallas kernel here")


# speedup vs baseline: 1.4799x; 1.4799x over previous
import jax
import jax.numpy as jnp
from jax.experimental import pallas as pl
from jax.experimental.pallas import tpu as pltpu

LOSS_COEF = 0.01


def _router_body(x_ref, wg_ref, gates_ref, loss_ref):
    x = x_ref[...]
    wg = wg_ref[...]
    n, e_dim = x.shape[0], wg.shape[1]
    logits = jnp.dot(x, wg, preferred_element_type=jnp.float32)
    eidx = jax.lax.broadcasted_iota(jnp.int32, (n, e_dim), 1)
    m1 = jnp.max(logits, axis=1, keepdims=True)
    a1 = jnp.min(jnp.where(logits == m1, eidx, e_dim), axis=1, keepdims=True)
    rest = jnp.where(eidx == a1, -jnp.inf, logits)
    m2 = jnp.max(rest, axis=1, keepdims=True)
    a2 = jnp.min(jnp.where(rest == m2, eidx, e_dim), axis=1, keepdims=True)
    z = jnp.exp(m2 - m1)
    denom = 1.0 + z
    g1 = 1.0 / denom
    g2 = z / denom
    gates = (jnp.where(eidx == a1, g1, 0.0) + jnp.where(eidx == a2, g2, 0.0))
    gates_ref[...] = gates
    importance = jnp.sum(gates, axis=0)
    load = jnp.sum((gates > 0.0).astype(jnp.float32), axis=0)

    def cv2(v):
        m = jnp.mean(v)
        var = jnp.sum((v - m) ** 2) / (e_dim - 1)
        return var / (m * m + 1e-10)

    loss_ref[0, 0] = (cv2(importance) + cv2(load)) * LOSS_COEF


def _expert_body(x_ref, w_ref, b_ref, g_ref, out_ref):
    e = pl.program_id(2)
    e_dim = b_ref.shape[0]
    x = x_ref[...]
    w = w_ref[0]
    acc = jnp.dot(x, w, preferred_element_type=jnp.float32)
    ridx = jax.lax.broadcasted_iota(jnp.int32, b_ref.shape, 0)
    brow = jnp.sum(jnp.where(ridx == e, b_ref[...], 0.0), axis=0,
                   keepdims=True)
    cidx = jax.lax.broadcasted_iota(jnp.int32, g_ref.shape, 1)
    gcol = jnp.sum(jnp.where(cidx == e, g_ref[...], 0.0), axis=1,
                   keepdims=True)
    term = gcol * (acc + brow)

    @pl.when(e == 0)
    def _():
        out_ref[...] = term

    @pl.when(e > 0)
    def _():
        out_ref[...] += term


def kernel(x, w_gate, W, b):
    orig_shape = x.shape[:-1]
    d = x.shape[-1]
    xf = x.reshape(-1, d)
    n = xf.shape[0]
    e_dim, _, h = W.shape

    gates, loss = pl.pallas_call(
        _router_body,
        out_shape=[
            jax.ShapeDtypeStruct((n, e_dim), jnp.float32),
            jax.ShapeDtypeStruct((1, 1), jnp.float32),
        ],
        out_specs=[
            pl.BlockSpec(memory_space=pltpu.VMEM),
            pl.BlockSpec(memory_space=pltpu.SMEM),
        ],
    )(xf, w_gate)

    tn = 1024
    th = 512
    grid = (n // tn, h // th, e_dim)
    y = pl.pallas_call(
        _expert_body,
        grid=grid,
        in_specs=[
            pl.BlockSpec((tn, d), lambda t, hh, e: (t, 0)),
            pl.BlockSpec((1, d, th), lambda t, hh, e: (e, 0, hh)),
            pl.BlockSpec((e_dim, th), lambda t, hh, e: (0, hh)),
            pl.BlockSpec((tn, e_dim), lambda t, hh, e: (t, 0)),
        ],
        out_specs=pl.BlockSpec((tn, th), lambda t, hh, e: (t, hh)),
        out_shape=jax.ShapeDtypeStruct((n, h), jnp.float32),
        compiler_params=pltpu.CompilerParams(
            dimension_semantics=("parallel", "parallel", "arbitrary"),
        ),
    )(xf, W, b, gates)

    return y.reshape(orig_shape + (h,)), loss[0, 0]


# speedup vs baseline: 1.4826x; 1.0018x over previous
import jax
import jax.numpy as jnp
from jax.experimental import pallas as pl
from jax.experimental.pallas import tpu as pltpu

LOSS_COEF = 0.01


def _router_body(x_ref, wg_ref, gates_ref, loss_ref):
    x = x_ref[...]
    wg = wg_ref[...]
    n, e_dim = x.shape[0], wg.shape[1]
    logits = jnp.dot(x, wg, preferred_element_type=jnp.float32)
    eidx = jax.lax.broadcasted_iota(jnp.int32, (n, e_dim), 1)
    m1 = jnp.max(logits, axis=1, keepdims=True)
    a1 = jnp.min(jnp.where(logits == m1, eidx, e_dim), axis=1, keepdims=True)
    rest = jnp.where(eidx == a1, -jnp.inf, logits)
    m2 = jnp.max(rest, axis=1, keepdims=True)
    a2 = jnp.min(jnp.where(rest == m2, eidx, e_dim), axis=1, keepdims=True)
    z = jnp.exp(m2 - m1)
    denom = 1.0 + z
    g1 = 1.0 / denom
    g2 = z / denom
    gates = (jnp.where(eidx == a1, g1, 0.0) + jnp.where(eidx == a2, g2, 0.0))
    gates_ref[...] = gates
    importance = jnp.sum(gates, axis=0)
    load = jnp.sum((gates > 0.0).astype(jnp.float32), axis=0)

    def cv2(v):
        m = jnp.mean(v)
        var = jnp.sum((v - m) ** 2) / (e_dim - 1)
        return var / (m * m + 1e-10)

    loss_ref[0, 0] = (cv2(importance) + cv2(load)) * LOSS_COEF


def _expert_body(x_ref, w_ref, b_ref, g_ref, out_ref):
    e = pl.program_id(2)
    e_dim = b_ref.shape[0]
    x = x_ref[...].astype(jnp.bfloat16)
    w = w_ref[0].astype(jnp.bfloat16)
    acc = jnp.dot(x, w, preferred_element_type=jnp.float32)
    ridx = jax.lax.broadcasted_iota(jnp.int32, b_ref.shape, 0)
    brow = jnp.sum(jnp.where(ridx == e, b_ref[...], 0.0), axis=0,
                   keepdims=True)
    cidx = jax.lax.broadcasted_iota(jnp.int32, g_ref.shape, 1)
    gcol = jnp.sum(jnp.where(cidx == e, g_ref[...], 0.0), axis=1,
                   keepdims=True)
    term = gcol * (acc + brow)

    @pl.when(e == 0)
    def _():
        out_ref[...] = term

    @pl.when(e > 0)
    def _():
        out_ref[...] += term


def kernel(x, w_gate, W, b):
    orig_shape = x.shape[:-1]
    d = x.shape[-1]
    xf = x.reshape(-1, d)
    n = xf.shape[0]
    e_dim, _, h = W.shape

    gates, loss = pl.pallas_call(
        _router_body,
        out_shape=[
            jax.ShapeDtypeStruct((n, e_dim), jnp.float32),
            jax.ShapeDtypeStruct((1, 1), jnp.float32),
        ],
        out_specs=[
            pl.BlockSpec(memory_space=pltpu.VMEM),
            pl.BlockSpec(memory_space=pltpu.SMEM),
        ],
    )(xf, w_gate)

    tn = 1024
    th = 512
    grid = (n // tn, h // th, e_dim)
    y = pl.pallas_call(
        _expert_body,
        grid=grid,
        in_specs=[
            pl.BlockSpec((tn, d), lambda t, hh, e: (t, 0)),
            pl.BlockSpec((1, d, th), lambda t, hh, e: (e, 0, hh)),
            pl.BlockSpec((e_dim, th), lambda t, hh, e: (0, hh)),
            pl.BlockSpec((tn, e_dim), lambda t, hh, e: (t, 0)),
        ],
        out_specs=pl.BlockSpec((tn, th), lambda t, hh, e: (t, hh)),
        out_shape=jax.ShapeDtypeStruct((n, h), jnp.float32),
        compiler_params=pltpu.CompilerParams(
            dimension_semantics=("parallel", "parallel", "arbitrary"),
        ),
    )(xf, W, b, gates)

    return y.reshape(orig_shape + (h,)), loss[0, 0]


# speedup vs baseline: 1.7232x; 1.1623x over previous
import jax
import jax.numpy as jnp
from jax.experimental import pallas as pl
from jax.experimental.pallas import tpu as pltpu

LOSS_COEF = 0.01


def _router_body(x_ref, wg_ref, gates_ref, loss_ref):
    x = x_ref[...]
    wg = wg_ref[...]
    n, e_dim = x.shape[0], wg.shape[1]
    logits = jnp.dot(x, wg, preferred_element_type=jnp.float32)
    eidx = jax.lax.broadcasted_iota(jnp.int32, (n, e_dim), 1)
    m1 = jnp.max(logits, axis=1, keepdims=True)
    a1 = jnp.min(jnp.where(logits == m1, eidx, e_dim), axis=1, keepdims=True)
    rest = jnp.where(eidx == a1, -jnp.inf, logits)
    m2 = jnp.max(rest, axis=1, keepdims=True)
    a2 = jnp.min(jnp.where(rest == m2, eidx, e_dim), axis=1, keepdims=True)
    z = jnp.exp(m2 - m1)
    denom = 1.0 + z
    g1 = 1.0 / denom
    g2 = z / denom
    gates = (jnp.where(eidx == a1, g1, 0.0) + jnp.where(eidx == a2, g2, 0.0))
    gates_ref[...] = gates
    importance = jnp.sum(gates, axis=0)
    load = jnp.sum((gates > 0.0).astype(jnp.float32), axis=0)

    def cv2(v):
        m = jnp.mean(v)
        var = jnp.sum((v - m) ** 2) / (e_dim - 1)
        return var / (m * m + 1e-10)

    loss_ref[0, 0] = (cv2(importance) + cv2(load)) * LOSS_COEF


def _expert_body(x_ref, w_ref, b_ref, g_ref, out_ref):
    e = pl.program_id(1)
    e_dim = b_ref.shape[0]
    x = x_ref[...].astype(jnp.bfloat16)
    w = w_ref[0].astype(jnp.bfloat16)
    acc = jnp.dot(x, w, preferred_element_type=jnp.float32)
    ridx = jax.lax.broadcasted_iota(jnp.int32, b_ref.shape, 0)
    brow = jnp.sum(jnp.where(ridx == e, b_ref[...], 0.0), axis=0,
                   keepdims=True)
    cidx = jax.lax.broadcasted_iota(jnp.int32, g_ref.shape, 1)
    gcol = jnp.sum(jnp.where(cidx == e, g_ref[...], 0.0), axis=1,
                   keepdims=True)
    term = gcol * (acc + brow)

    @pl.when(e == 0)
    def _():
        out_ref[...] = term

    @pl.when(e > 0)
    def _():
        out_ref[...] += term


def kernel(x, w_gate, W, b):
    orig_shape = x.shape[:-1]
    d = x.shape[-1]
    xf = x.reshape(-1, d)
    n = xf.shape[0]
    e_dim, _, h = W.shape

    gates, loss = pl.pallas_call(
        _router_body,
        out_shape=[
            jax.ShapeDtypeStruct((n, e_dim), jnp.float32),
            jax.ShapeDtypeStruct((1, 1), jnp.float32),
        ],
        out_specs=[
            pl.BlockSpec(memory_space=pltpu.VMEM),
            pl.BlockSpec(memory_space=pltpu.SMEM),
        ],
    )(xf, w_gate)

    tn = n
    th = 512
    grid = (h // th, e_dim)
    y = pl.pallas_call(
        _expert_body,
        grid=grid,
        in_specs=[
            pl.BlockSpec((tn, d), lambda hh, e: (0, 0)),
            pl.BlockSpec((1, d, th), lambda hh, e: (e, 0, hh)),
            pl.BlockSpec((e_dim, th), lambda hh, e: (0, hh)),
            pl.BlockSpec((tn, e_dim), lambda hh, e: (0, 0)),
        ],
        out_specs=pl.BlockSpec((tn, th), lambda hh, e: (0, hh)),
        out_shape=jax.ShapeDtypeStruct((n, h), jnp.float32),
        compiler_params=pltpu.CompilerParams(
            dimension_semantics=("parallel", "arbitrary"),
        ),
    )(xf, W, b, gates)

    return y.reshape(orig_shape + (h,)), loss[0, 0]
